# baseline (speedup 1.0000x reference)
import functools

import jax
import jax.numpy as jnp
from jax import lax
from jax.experimental import pallas as pl
from jax.experimental.pallas import tpu as pltpu
from jax.experimental.pallas import tpu_sc as plsc

D = 100
DP = 128
NBUF = 3

_info = plsc.get_sparse_core_info()
_NC, _NS = _info.num_cores, _info.num_subcores
NW = _NC * _NS


def _emb_call(b, s):
    bpw = b // NW
    n_groups = bpw // NBUF
    assert n_groups >= 2
    n_total = b * s
    mesh = plsc.VectorSubcoreMesh(core_axis_name="c", subcore_axis_name="s")

    @functools.partial(
        pl.kernel,
        out_type=jax.ShapeDtypeStruct((n_total, DP), jnp.float32),
        mesh=mesh,
        scratch_types=[
            pltpu.VMEM((bpw, 256), jnp.int32),
            pltpu.VMEM((NBUF, s, DP), jnp.float32),
        ] + [pltpu.SemaphoreType.DMA] * (3 * NBUF),
        compiler_params=pltpu.CompilerParams(use_tc_tiling_on_sc=True),
    )
    def emb(x_hbm, table_hbm, out_hbm, xv, rows_v, *sems):
        wid = lax.axis_index("s") * _NC + lax.axis_index("c")
        b0 = wid * bpw
        ga = sems[:NBUF]
        gb = sems[NBUF:2 * NBUF]
        os_ = sems[2 * NBUF:]
        pltpu.sync_copy(x_hbm.at[pl.ds(b0, bpw)], xv)

        def gather_a(r, slot):
            return pltpu.make_async_copy(
                table_hbm.at[xv.at[r, pl.ds(0, 128)]],
                rows_v.at[slot].at[pl.ds(0, 128)], ga[slot])

        def gather_b(r, slot):
            return pltpu.make_async_copy(
                table_hbm.at[xv.at[r, pl.ds(128, s - 128)]],
                rows_v.at[slot].at[pl.ds(128, s - 128)], gb[slot])

        def put(r, slot):
            return pltpu.make_async_copy(
                rows_v.at[slot], out_hbm.at[pl.ds((b0 + r) * s, s)], os_[slot])

        for slot in range(NBUF):
            gather_a(slot, slot).start()
            gather_b(slot, slot).start()

        def body(i, carry):
            r0 = i * NBUF
            for slot in range(NBUF):
                gather_a(r0 + slot, slot).wait()
                gather_b(r0 + slot, slot).wait()
                put(r0 + slot, slot).start()
            for slot in range(NBUF):
                put(r0 + slot, slot).wait()
                gather_a(r0 + NBUF + slot, slot).start()
                gather_b(r0 + NBUF + slot, slot).start()
            return carry

        lax.fori_loop(0, n_groups - 1, body, 0)

        for r in range((n_groups - 1) * NBUF, bpw):
            slot = r % NBUF
            gather_a(r, slot).wait()
            gather_b(r, slot).wait()
            put(r, slot).start()
            nr = r + NBUF
            if nr < bpw:
                put(r, slot).wait()
                gather_a(nr, slot).start()
                gather_b(nr, slot).start()
        for r in range(bpw - NBUF, bpw):
            put(r, r % NBUF).wait()

    return emb


def kernel(x, table):
    b, s = x.shape
    x_p = jnp.pad(x, ((0, 0), (0, 256 - s)))
    table_p = jnp.pad(table, ((0, 0), (0, DP - D)))
    out = _emb_call(b, s)(x_p, table_p)
    return out[:, :D].reshape(b, s, D)


# speedup vs baseline: 1.0020x; 1.0020x over previous
import functools

import jax
import jax.numpy as jnp
from jax import lax
from jax.experimental import pallas as pl
from jax.experimental.pallas import tpu as pltpu
from jax.experimental.pallas import tpu_sc as plsc

D = 100
DP = 128
NBUF = 3

_info = plsc.get_sparse_core_info()
_NC, _NS = _info.num_cores, _info.num_subcores
NW = _NC * _NS


def _emb_call(b, s):
    bpw = b // NW
    n_groups = bpw // NBUF
    assert n_groups >= 2 and s > 128
    n_total = b * s
    mesh = plsc.VectorSubcoreMesh(core_axis_name="c", subcore_axis_name="s")

    @functools.partial(
        pl.kernel,
        out_type=jax.ShapeDtypeStruct((n_total, DP), jnp.float32),
        mesh=mesh,
        scratch_types=[
            pltpu.VMEM((bpw, 128), jnp.int32),
            pltpu.VMEM((bpw, 128), jnp.int32),
            pltpu.VMEM((NBUF, s, DP), jnp.float32),
        ] + [pltpu.SemaphoreType.DMA] * (3 * NBUF),
        compiler_params=pltpu.CompilerParams(
            use_tc_tiling_on_sc=True, needs_layout_passes=False),
    )
    def emb(xa_hbm, xb_hbm, table_hbm, out_hbm, xa_v, xb_v, rows_v, *sems):
        wid = lax.axis_index("s") * _NC + lax.axis_index("c")
        b0 = wid * bpw
        ga = sems[:NBUF]
        gb = sems[NBUF:2 * NBUF]
        os_ = sems[2 * NBUF:]
        pltpu.sync_copy(xa_hbm.at[pl.ds(b0, bpw)], xa_v)
        pltpu.sync_copy(xb_hbm.at[pl.ds(b0, bpw)], xb_v)

        def gather_a(r, slot):
            return pltpu.make_async_copy(
                table_hbm.at[xa_v.at[r]],
                rows_v.at[slot].at[pl.ds(0, 128)], ga[slot])

        def gather_b(r, slot):
            return pltpu.make_async_copy(
                table_hbm.at[xb_v.at[r, pl.ds(0, s - 128)]],
                rows_v.at[slot].at[pl.ds(128, s - 128)], gb[slot])

        def put(r, slot):
            return pltpu.make_async_copy(
                rows_v.at[slot], out_hbm.at[pl.ds((b0 + r) * s, s)], os_[slot])

        for slot in range(NBUF):
            gather_a(slot, slot).start()
            gather_b(slot, slot).start()

        def body(i, carry):
            r0 = i * NBUF
            for slot in range(NBUF):
                gather_a(r0 + slot, slot).wait()
                gather_b(r0 + slot, slot).wait()
                put(r0 + slot, slot).start()
            for slot in range(NBUF):
                put(r0 + slot, slot).wait()
                gather_a(r0 + NBUF + slot, slot).start()
                gather_b(r0 + NBUF + slot, slot).start()
            return carry

        lax.fori_loop(0, n_groups - 1, body, 0)

        for r in range((n_groups - 1) * NBUF, bpw):
            slot = r % NBUF
            gather_a(r, slot).wait()
            gather_b(r, slot).wait()
            put(r, slot).start()
            nr = r + NBUF
            if nr < bpw:
                put(r, slot).wait()
                gather_a(nr, slot).start()
                gather_b(nr, slot).start()
        for r in range(bpw - NBUF, bpw):
            put(r, r % NBUF).wait()

    return emb


def kernel(x, table):
    b, s = x.shape
    xa = lax.slice(x, (0, 0), (b, 128))
    xb = jnp.pad(lax.slice(x, (0, 128), (b, s)), ((0, 0), (0, 256 - s)))
    table_p = jnp.pad(table, ((0, 0), (0, DP - D)))
    out = _emb_call(b, s)(xa, xb, table_p)
    return out[:, :D].reshape(b, s, D)
